# baseline (speedup 1.0000x reference)
import functools
import jax
import jax.numpy as jnp
from jax import lax
from jax.experimental import pallas as pl
from jax.experimental.pallas import tpu as pltpu
from jax.experimental.pallas import tpu_sc as plsc

N = 10000
NPAD = 10240
E = 320000
D = 128

NC = 2
NS = 16
NW = NC * NS
EPT = E // NW
K = 80
NCH = EPT // K
EPTP = EPT
NSLOT = 4
RPT = NPAD // NS

_MESH = plsc.VectorSubcoreMesh(core_axis_name="c", subcore_axis_name="s")
_SC_PARAMS = pltpu.CompilerParams(needs_layout_passes=False)


@functools.partial(
    pl.kernel,
    out_type=jax.ShapeDtypeStruct((NW * NPAD,), jnp.float32),
    mesh=_MESH,
    scratch_types=[
        pltpu.VMEM((EPT,), jnp.int32),
        pltpu.VMEM((NPAD,), jnp.float32),
    ],
    compiler_params=_SC_PARAMS,
)
def _deg_kernel(dst_hbm, deg_out, idx_v, deg_v):
    c = lax.axis_index("c")
    s = lax.axis_index("s")
    wid = c * NS + s
    base = wid * EPT
    pltpu.sync_copy(dst_hbm.at[pl.ds(base, EPT)], idx_v)

    zeros = jnp.zeros((16,), jnp.float32)
    ones = jnp.ones((16,), jnp.float32)

    def zbody(i, carry):
        for u in range(5):
            deg_v[pl.ds((i * 5 + u) * 16, 16)] = zeros
        return carry

    lax.fori_loop(0, NPAD // 16 // 5, zbody, 0)

    def sbody(i, carry):
        for u in range(5):
            idx = idx_v[pl.ds((i * 5 + u) * 16, 16)]
            plsc.addupdate_scatter(deg_v, [idx], ones)
        return carry

    lax.fori_loop(0, EPT // 16 // 5, sbody, 0)
    pltpu.sync_copy(deg_v, deg_out.at[pl.ds(wid * NPAD, NPAD)])


@functools.partial(
    pl.kernel,
    out_type=jax.ShapeDtypeStruct((NC, NPAD, D), jnp.float32),
    mesh=_MESH,
    scratch_types=[
        [pltpu.VMEM((K,), jnp.int32)] * NSLOT,
        [pltpu.VMEM((K,), jnp.int32)] * NSLOT,
        [pltpu.VMEM((K, D), jnp.float32)] * NSLOT,
        pltpu.VMEM_SHARED((NPAD, D), jnp.float32),
        [pltpu.SemaphoreType.DMA] * NSLOT,
        [pltpu.SemaphoreType.DMA] * NSLOT,
        [pltpu.SemaphoreType.DMA] * NSLOT,
        [pltpu.SemaphoreType.DMA] * NSLOT,
    ],
    compiler_params=_SC_PARAMS,
)
def _agg_kernel(hs_hbm, src_hbm, dst_hbm, part_out, sidx, didx, rows,
                acc_sh, sem_si, sem_d, sem_g, sem_s):
    c = lax.axis_index("c")
    s = lax.axis_index("s")
    wid = c * NS + s
    base = wid * EPTP
    io_chunks = [(i * K, K) for i in range(RPT // K)]
    if RPT % K:
        io_chunks.append((RPT // K * K, RPT % K))

    zeros = jnp.zeros((16,), jnp.float32)

    def zbody(t, carry):
        rows[0][t // (D // 16), pl.ds((t % (D // 16)) * 16, 16)] = zeros
        return carry

    lax.fori_loop(0, K * (D // 16), zbody, 0)
    for off, ln in io_chunks:
        pltpu.sync_copy(rows[0].at[pl.ds(0, ln), :],
                        acc_sh.at[pl.ds(s * RPT + off, ln), :])
    plsc.subcore_barrier()

    def sidx_cp(t, b):
        return pltpu.make_async_copy(src_hbm.at[pl.ds(base + t * K, K)],
                                     sidx[b], sem_si[b])

    def didx_cp(t, b):
        return pltpu.make_async_copy(dst_hbm.at[pl.ds(base + t * K, K)],
                                     didx[b], sem_d[b])

    for t0 in range(2):
        sidx_cp(t0, t0).start()

    def body(g, carry):
        for b in range(NSLOT):
            t = g * NSLOT + b
            bp = (b + 2) % NSLOT

            @pl.when(jnp.logical_and(t >= NSLOT, t < NCH + NSLOT))
            def _():
                pltpu.make_async_copy(rows[b], acc_sh.at[didx[b]],
                                      sem_s[b]).wait()

            @pl.when(t < NCH)
            def _():
                didx_cp(t, b).start()

            @pl.when(t < NCH)
            def _():
                sidx_cp(t, b).wait()
                pltpu.async_copy(hs_hbm.at[sidx[b]], rows[b], sem_g[b])

            @pl.when(jnp.logical_and(t >= 2, t < NCH + 2))
            def _():
                pltpu.make_async_copy(hs_hbm.at[sidx[bp]], rows[bp],
                                      sem_g[bp]).wait()
                didx_cp(t - 2, bp).wait()
                pltpu.async_copy(rows[bp], acc_sh.at[didx[bp]], sem_s[bp],
                                 add=True)

            @pl.when(t + 2 < NCH)
            def _():
                sidx_cp(t + 2, bp).start()

        return carry

    lax.fori_loop(0, (NCH + NSLOT + NSLOT - 1) // NSLOT, body, 0)
    plsc.subcore_barrier()

    for off, ln in io_chunks:
        r0 = s * RPT + off
        pltpu.sync_copy(acc_sh.at[pl.ds(r0, ln), :],
                        part_out.at[c, pl.ds(r0, ln), :])


def _dinv_body(degp_ref, dinv_ref):
    deg = jnp.sum(degp_ref[...], axis=0) + 1.0
    dinv_ref[...] = lax.rsqrt(deg)[:, None]


def _mm_body(x_ref, w_ref, dinv_ref, hs_ref):
    h = jnp.dot(x_ref[...], w_ref[...], preferred_element_type=jnp.float32)
    hs_ref[...] = h * dinv_ref[...]


def _out_body(p_ref, hs_ref, dinv_ref, b_ref, o_ref):
    v = (p_ref[0] + p_ref[1] + hs_ref[...]) * dinv_ref[...] + b_ref[...]
    m = jnp.max(v, axis=1, keepdims=True)
    z = v - m
    o_ref[...] = z - jnp.log(jnp.sum(jnp.exp(z), axis=1, keepdims=True))


_BN = 5000


def kernel(x, edge_index, W, b):
    src = edge_index[0]
    dst = edge_index[1]
    degp = _deg_kernel(dst).reshape(NW, NPAD)

    dinv = pl.pallas_call(
        _dinv_body,
        grid=(1,),
        in_specs=[pl.BlockSpec((NW, NPAD), lambda i: (0, 0))],
        out_specs=pl.BlockSpec((NPAD, 1), lambda i: (0, 0)),
        out_shape=jax.ShapeDtypeStruct((NPAD, 1), jnp.float32),
    )(degp)

    hs = pl.pallas_call(
        _mm_body,
        grid=(N // _BN,),
        in_specs=[
            pl.BlockSpec((_BN, D), lambda i: (i, 0)),
            pl.BlockSpec((D, D), lambda i: (0, 0)),
            pl.BlockSpec((_BN, 1), lambda i: (i, 0)),
        ],
        out_specs=pl.BlockSpec((_BN, D), lambda i: (i, 0)),
        out_shape=jax.ShapeDtypeStruct((N, D), jnp.float32),
    )(x, W, dinv)

    parts = _agg_kernel(hs, src, dst)

    out = pl.pallas_call(
        _out_body,
        grid=(N // _BN,),
        in_specs=[
            pl.BlockSpec((NC, _BN, D), lambda i: (0, i, 0)),
            pl.BlockSpec((_BN, D), lambda i: (i, 0)),
            pl.BlockSpec((_BN, 1), lambda i: (i, 0)),
            pl.BlockSpec((1, D), lambda i: (0, 0)),
        ],
        out_specs=pl.BlockSpec((_BN, D), lambda i: (i, 0)),
        out_shape=jax.ShapeDtypeStruct((N, D), jnp.float32),
    )(parts, hs, dinv, b.reshape(1, D))

    return out


# speedup vs baseline: 1.0165x; 1.0165x over previous
import functools
import jax
import jax.numpy as jnp
from jax import lax
from jax.experimental import pallas as pl
from jax.experimental.pallas import tpu as pltpu
from jax.experimental.pallas import tpu_sc as plsc

N = 10000
NPAD = 10240
E = 320000
D = 128

NC = 2
NS = 16
NW = NC * NS
EPT = E // NW
K = 80
NCH = EPT // K
EPTP = EPT
NSLOT = 4
RPT = NPAD // NS

_MESH = plsc.VectorSubcoreMesh(core_axis_name="c", subcore_axis_name="s")
_SC_PARAMS = pltpu.CompilerParams(needs_layout_passes=False)


@functools.partial(
    pl.kernel,
    out_type=jax.ShapeDtypeStruct((NW * NPAD,), jnp.float32),
    mesh=_MESH,
    scratch_types=[
        pltpu.VMEM((EPT,), jnp.int32),
        pltpu.VMEM((NPAD,), jnp.float32),
    ],
    compiler_params=_SC_PARAMS,
)
def _deg_kernel(dst_hbm, deg_out, idx_v, deg_v):
    c = lax.axis_index("c")
    s = lax.axis_index("s")
    wid = c * NS + s
    base = wid * EPT
    pltpu.sync_copy(dst_hbm.at[pl.ds(base, EPT)], idx_v)

    zeros = jnp.zeros((16,), jnp.float32)
    ones = jnp.ones((16,), jnp.float32)

    def zbody(i, carry):
        for u in range(5):
            deg_v[pl.ds((i * 5 + u) * 16, 16)] = zeros
        return carry

    lax.fori_loop(0, NPAD // 16 // 5, zbody, 0)

    def sbody(i, carry):
        for u in range(5):
            idx = idx_v[pl.ds((i * 5 + u) * 16, 16)]
            plsc.addupdate_scatter(deg_v, [idx], ones)
        return carry

    lax.fori_loop(0, EPT // 16 // 5, sbody, 0)
    pltpu.sync_copy(deg_v, deg_out.at[pl.ds(wid * NPAD, NPAD)])


@functools.partial(
    pl.kernel,
    out_type=jax.ShapeDtypeStruct((NC, NPAD, D), jnp.float32),
    mesh=_MESH,
    scratch_types=[
        [pltpu.VMEM((K,), jnp.int32)] * NSLOT,
        [pltpu.VMEM((K,), jnp.int32)] * NSLOT,
        [pltpu.VMEM((K, D), jnp.float32)] * NSLOT,
        pltpu.VMEM_SHARED((NPAD, D), jnp.float32),
        [pltpu.SemaphoreType.DMA] * NSLOT,
        [pltpu.SemaphoreType.DMA] * NSLOT,
        [pltpu.SemaphoreType.DMA] * NSLOT,
        [pltpu.SemaphoreType.DMA] * NSLOT,
    ],
    compiler_params=_SC_PARAMS,
)
def _agg_kernel(hs_hbm, src_hbm, dst_hbm, part_out, sidx, didx, rows,
                acc_sh, sem_si, sem_d, sem_g, sem_s):
    c = lax.axis_index("c")
    s = lax.axis_index("s")
    wid = c * NS + s
    base = wid * EPTP
    io_chunks = [(i * K, K) for i in range(RPT // K)]
    if RPT % K:
        io_chunks.append((RPT // K * K, RPT % K))

    zeros = jnp.zeros((16,), jnp.float32)

    def zbody(t, carry):
        rows[0][t // (D // 16), pl.ds((t % (D // 16)) * 16, 16)] = zeros
        return carry

    lax.fori_loop(0, K * (D // 16), zbody, 0)
    for off, ln in io_chunks:
        pltpu.sync_copy(rows[0].at[pl.ds(0, ln), :],
                        acc_sh.at[pl.ds(s * RPT + off, ln), :])
    plsc.subcore_barrier()

    def sidx_cp(t, b):
        return pltpu.make_async_copy(src_hbm.at[pl.ds(base + t * K, K)],
                                     sidx[b], sem_si[b])

    def didx_cp(t, b):
        return pltpu.make_async_copy(dst_hbm.at[pl.ds(base + t * K, K)],
                                     didx[b], sem_d[b])

    for t0 in range(2):
        sidx_cp(t0, t0).start()

    def body(g, carry):
        for b in range(NSLOT):
            t = g * NSLOT + b
            bp = (b + 2) % NSLOT

            @pl.when(jnp.logical_and(t >= NSLOT, t < NCH + NSLOT))
            def _():
                pltpu.make_async_copy(rows[b], acc_sh.at[didx[b]],
                                      sem_s[b]).wait()

            @pl.when(t < NCH)
            def _():
                didx_cp(t, b).start()

            @pl.when(t < NCH)
            def _():
                sidx_cp(t, b).wait()
                pltpu.async_copy(hs_hbm.at[sidx[b]], rows[b], sem_g[b])

            @pl.when(jnp.logical_and(t >= 2, t < NCH + 2))
            def _():
                pltpu.make_async_copy(hs_hbm.at[sidx[bp]], rows[bp],
                                      sem_g[bp]).wait()
                didx_cp(t - 2, bp).wait()
                pltpu.async_copy(rows[bp], acc_sh.at[didx[bp]], sem_s[bp],
                                 add=True)

            @pl.when(t + 2 < NCH)
            def _():
                sidx_cp(t + 2, bp).start()

        return carry

    lax.fori_loop(0, (NCH + NSLOT + NSLOT - 1) // NSLOT, body, 0)
    plsc.subcore_barrier()

    for off, ln in io_chunks:
        r0 = s * RPT + off
        pltpu.sync_copy(acc_sh.at[pl.ds(r0, ln), :],
                        part_out.at[c, pl.ds(r0, ln), :])


def _mm_body(x_ref, w_ref, degp_ref, hs_ref):
    deg = jnp.sum(degp_ref[...], axis=1) + 1.0
    dinv = lax.rsqrt(deg)
    h = jnp.dot(x_ref[...], w_ref[...], preferred_element_type=jnp.float32)
    hs_ref[...] = h * dinv[:, None]


def _out_body(p_ref, hs_ref, degp_ref, b_ref, o_ref):
    deg = jnp.sum(degp_ref[...], axis=1) + 1.0
    dinv = lax.rsqrt(deg)
    v = (p_ref[0] + p_ref[1] + hs_ref[...]) * dinv[:, None] + b_ref[...]
    m = jnp.max(v, axis=1, keepdims=True)
    z = v - m
    o_ref[...] = z - jnp.log(jnp.sum(jnp.exp(z), axis=1, keepdims=True))


_BN = 5000


def kernel(x, edge_index, W, b):
    src = edge_index[0]
    dst = edge_index[1]
    degp = _deg_kernel(dst).reshape(NW, NPAD).T

    hs = pl.pallas_call(
        _mm_body,
        grid=(N // _BN,),
        in_specs=[
            pl.BlockSpec((_BN, D), lambda i: (i, 0)),
            pl.BlockSpec((D, D), lambda i: (0, 0)),
            pl.BlockSpec((_BN, NW), lambda i: (i, 0)),
        ],
        out_specs=pl.BlockSpec((_BN, D), lambda i: (i, 0)),
        out_shape=jax.ShapeDtypeStruct((N, D), jnp.float32),
    )(x, W, degp)

    parts = _agg_kernel(hs, src, dst)

    out = pl.pallas_call(
        _out_body,
        grid=(N // _BN,),
        in_specs=[
            pl.BlockSpec((NC, _BN, D), lambda i: (0, i, 0)),
            pl.BlockSpec((_BN, D), lambda i: (i, 0)),
            pl.BlockSpec((_BN, NW), lambda i: (i, 0)),
            pl.BlockSpec((1, D), lambda i: (0, 0)),
        ],
        out_specs=pl.BlockSpec((_BN, D), lambda i: (i, 0)),
        out_shape=jax.ShapeDtypeStruct((N, D), jnp.float32),
    )(parts, hs, degp, b.reshape(1, D))

    return out
